# Initial kernel scaffold; baseline (speedup 1.0000x reference)
import jax
import jax.numpy as jnp
from jax.experimental import pallas as pl
from jax.experimental.pallas import tpu as pltpu

N_EMBD = 768
SEG = N_EMBD // 3


def _body(time_ref, height_ref, width_ref, x_ref, o_ref):
    xb = x_ref[0, 0]
    t_row = time_ref[0]
    w_tab = width_ref[...]
    h_tab = height_ref[...]
    o_ref[0, 0, :, :, 0:SEG] = xb[:, :, 0:SEG] + t_row[None, None, :]
    o_ref[0, 0, :, :, SEG:2 * SEG] = xb[:, :, SEG:2 * SEG] + w_tab[None, :, :]
    o_ref[0, 0, :, :, 2 * SEG:3 * SEG] = xb[:, :, 2 * SEG:3 * SEG] + h_tab[:, None, :]


def kernel(x, time_embed, height_embed, width_embed):
    B, t, h, w, d = x.shape
    grid = (B, t)
    return pl.pallas_call(
        _body,
        grid=grid,
        in_specs=[
            pl.BlockSpec((1, SEG), lambda b, ti: (ti, 0)),
            pl.BlockSpec((h, SEG), lambda b, ti: (0, 0)),
            pl.BlockSpec((w, SEG), lambda b, ti: (0, 0)),
            pl.BlockSpec((1, 1, h, w, d), lambda b, ti: (b, ti, 0, 0, 0)),
        ],
        out_specs=pl.BlockSpec((1, 1, h, w, d), lambda b, ti: (b, ti, 0, 0, 0)),
        out_shape=jax.ShapeDtypeStruct(x.shape, x.dtype),
    )(time_embed, height_embed, width_embed, x)


# speedup vs baseline: 1.0941x; 1.0941x over previous
import jax
import jax.numpy as jnp
from jax.experimental import pallas as pl
from jax.experimental.pallas import tpu as pltpu

N_EMBD = 768
SEG = N_EMBD // 3


def _body(time_ref, height_ref, width_ref, x_ref, o_ref):
    ti = pl.program_id(1)
    xb = x_ref[0, 0]
    t_row = time_ref[pl.ds(ti, 1), :][0]
    w_tab = width_ref[...]
    h_tab = height_ref[...]
    o_ref[0, 0, :, :, 0:SEG] = xb[:, :, 0:SEG] + t_row[None, None, :]
    o_ref[0, 0, :, :, SEG:2 * SEG] = xb[:, :, SEG:2 * SEG] + w_tab[None, :, :]
    o_ref[0, 0, :, :, 2 * SEG:3 * SEG] = xb[:, :, 2 * SEG:3 * SEG] + h_tab[:, None, :]


def kernel(x, time_embed, height_embed, width_embed):
    B, t, h, w, d = x.shape
    grid = (B, t)
    return pl.pallas_call(
        _body,
        grid=grid,
        in_specs=[
            pl.BlockSpec((t, SEG), lambda b, ti: (0, 0)),
            pl.BlockSpec((h, SEG), lambda b, ti: (0, 0)),
            pl.BlockSpec((w, SEG), lambda b, ti: (0, 0)),
            pl.BlockSpec((1, 1, h, w, d), lambda b, ti: (b, ti, 0, 0, 0)),
        ],
        out_specs=pl.BlockSpec((1, 1, h, w, d), lambda b, ti: (b, ti, 0, 0, 0)),
        out_shape=jax.ShapeDtypeStruct(x.shape, x.dtype),
    )(time_embed, height_embed, width_embed, x)
